# baseline (speedup 1.0000x reference)
import functools

import jax
import jax.numpy as jnp
from jax.experimental import pallas as pl
from jax.experimental.pallas import tpu as pltpu
from jax.experimental.pallas import tpu_sc as plsc

_MT = 1024
_NT = 1024
_GW = 128
_FW = 128


def _argmin_body(
    nc_tiles, x_ref, c_ref, idx_ref, loss_ref, minv_ref, mini_ref, xn_ref, cn_ref, x2_ref
):
    i = pl.program_id(0)
    j = pl.program_id(1)
    c = c_ref[...]

    @pl.when(j == 0)
    def _():
        x = x_ref[...]
        xn_ref[...] = jnp.sum(x * x, axis=1, keepdims=True)
        x2_ref[...] = x + x

    @pl.when(i == 0)
    def _():
        cn_ref[j, :] = jnp.sum(c * c, axis=1)

    xnorm = xn_ref[...]
    cnorm = cn_ref[j, :]
    prod2 = jax.lax.dot_general(
        x2_ref[...], c, (((1,), (1,)), ((), ())), preferred_element_type=jnp.float32
    )
    d = (xnorm + cnorm[None, :]) - prod2

    v = d
    off = None
    half = _NT // 2
    while half >= _FW:
        lo = v[:, :half]
        hi = v[:, half:]
        take = hi < lo
        if off is None:
            off = jnp.where(take, jnp.float32(half), jnp.float32(0.0))
        else:
            off = jnp.where(take, off[:, half:] + jnp.float32(half), off[:, :half])
        v = jnp.minimum(lo, hi)
        half //= 2
    base = off + jnp.float32(j * _NT)

    @pl.when(j == 0)
    def _():
        minv_ref[...] = v
        mini_ref[...] = base

    @pl.when(j != 0)
    def _():
        prev_v = minv_ref[...]
        take2 = v < prev_v
        minv_ref[...] = jnp.minimum(v, prev_v)
        mini_ref[...] = jnp.where(take2, base, mini_ref[...])

    @pl.when(j == nc_tiles - 1)
    def _():
        vv = minv_ref[...]
        best = jnp.min(vv, axis=1)
        full_idx = mini_ref[...] + jax.lax.broadcasted_iota(
            jnp.int32, (_MT, _FW), 1
        ).astype(jnp.float32)
        cand = jnp.where(vv == best[:, None], full_idx, jnp.float32(2.0**30))
        idx_ref[...] = jnp.min(cand, axis=1, keepdims=True).astype(jnp.int32)
        tile_loss = jnp.sum(best)
        prev = jnp.where(i == 0, 0.0, loss_ref[0, 0])
        loss_ref[0, 0] = prev + tile_loss


def _tc_argmin(flat, codebook):
    m, k = flat.shape
    nc = codebook.shape[0]
    m_tiles, nc_tiles = m // _MT, nc // _NT
    idx, loss_sum = pl.pallas_call(
        functools.partial(_argmin_body, nc_tiles),
        grid=(m_tiles, nc_tiles),
        in_specs=[
            pl.BlockSpec((_MT, k), lambda i, j: (i, 0)),
            pl.BlockSpec((_NT, k), lambda i, j: (j, 0)),
        ],
        out_specs=[
            pl.BlockSpec((_MT, 1), lambda i, j: (i, 0)),
            pl.BlockSpec(memory_space=pltpu.SMEM),
        ],
        out_shape=[
            jax.ShapeDtypeStruct((m, 1), jnp.int32),
            jax.ShapeDtypeStruct((1, 1), jnp.float32),
        ],
        scratch_shapes=[
            pltpu.VMEM((_MT, _FW), jnp.float32),
            pltpu.VMEM((_MT, _FW), jnp.float32),
            pltpu.VMEM((_MT, 1), jnp.float32),
            pltpu.VMEM((nc_tiles, _NT), jnp.float32),
            pltpu.VMEM((_MT, k), jnp.float32),
        ],
    )(flat, codebook)
    return idx.reshape(m), loss_sum[0, 0]


def _sc_gather(codebook, indices):
    t = indices.shape[0]
    k = codebook.shape[1]
    idx2 = indices.reshape(1, t)
    mesh = plsc.VectorSubcoreMesh(core_axis_name="core", subcore_axis_name="subcore")

    @functools.partial(
        pl.kernel,
        out_type=jax.ShapeDtypeStruct((t, k), codebook.dtype),
        mesh=mesh,
    )
    def gather_kernel(x_hbm, i_hbm, o_hbm):
        def body(i_vmem, o_vmem):
            pltpu.sync_copy(x_hbm.at[i_vmem.at[0]], o_vmem)

        pltpu.emit_pipeline(
            body,
            grid=(t // _GW,),
            in_specs=[pl.BlockSpec((1, _GW), index_map=lambda i: (0, i))],
            out_specs=[pl.BlockSpec((_GW, k), index_map=lambda i: (i, 0))],
            core_axis_name=("core", "subcore"),
            dimension_semantics=(pltpu.PARALLEL,),
        )(i_hbm, o_hbm)

    return gather_kernel(codebook, idx2)


def kernel(z_e, codebook):
    n, c, d, h, w = z_e.shape
    t = n * d * h * w
    flat = jnp.transpose(z_e, (0, 2, 3, 4, 1)).reshape(t, c)
    indices, loss_sum = _tc_argmin(flat, codebook)
    z_q_flat = _sc_gather(codebook, indices)
    vq_loss = 0.5 * loss_sum / (t * c)
    z_q = jnp.transpose(z_q_flat.reshape(n, d, h, w, c), (0, 4, 1, 2, 3))
    return (z_q, vq_loss, indices.reshape(n, d, h, w))


# speedup vs baseline: 1.0834x; 1.0834x over previous
import functools

import jax
import jax.numpy as jnp
from jax.experimental import pallas as pl
from jax.experimental.pallas import tpu as pltpu
from jax.experimental.pallas import tpu_sc as plsc

_MT = 2048
_NT = 1024
_GW = 128
_FW = 128


def _argmin_body(
    nc_tiles, x_ref, c_ref, idx_ref, loss_ref, minv_ref, mini_ref, xn_ref, cn_ref, x2_ref
):
    i = pl.program_id(0)
    j = pl.program_id(1)
    c = c_ref[...]

    @pl.when(j == 0)
    def _():
        x = x_ref[...]
        xn_ref[...] = jnp.sum(x * x, axis=1, keepdims=True)
        x2_ref[...] = x + x

    @pl.when(i == 0)
    def _():
        cn_ref[j, :] = jnp.sum(c * c, axis=1)

    xnorm = xn_ref[...]
    cnorm = cn_ref[j, :]
    prod2 = jax.lax.dot_general(
        x2_ref[...], c, (((1,), (1,)), ((), ())), preferred_element_type=jnp.float32
    )
    d = (xnorm + cnorm[None, :]) - prod2

    v = d
    off = None
    half = _NT // 2
    while half >= _FW:
        lo = v[:, :half]
        hi = v[:, half:]
        take = hi < lo
        if off is None:
            off = jnp.where(take, jnp.float32(half), jnp.float32(0.0))
        else:
            off = jnp.where(take, off[:, half:] + jnp.float32(half), off[:, :half])
        v = jnp.minimum(lo, hi)
        half //= 2
    base = off + jnp.float32(j * _NT)

    @pl.when(j == 0)
    def _():
        minv_ref[...] = v
        mini_ref[...] = base

    @pl.when(j != 0)
    def _():
        prev_v = minv_ref[...]
        take2 = v < prev_v
        minv_ref[...] = jnp.minimum(v, prev_v)
        mini_ref[...] = jnp.where(take2, base, mini_ref[...])

    @pl.when(j == nc_tiles - 1)
    def _():
        vv = minv_ref[...]
        best = jnp.min(vv, axis=1)
        full_idx = mini_ref[...] + jax.lax.broadcasted_iota(
            jnp.int32, (_MT, _FW), 1
        ).astype(jnp.float32)
        cand = jnp.where(vv == best[:, None], full_idx, jnp.float32(2.0**30))
        idx_ref[...] = jnp.min(cand, axis=1, keepdims=True).astype(jnp.int32)
        tile_loss = jnp.sum(best)
        prev = jnp.where(i == 0, 0.0, loss_ref[0, 0])
        loss_ref[0, 0] = prev + tile_loss


def _tc_argmin(flat, codebook):
    m, k = flat.shape
    nc = codebook.shape[0]
    m_tiles, nc_tiles = m // _MT, nc // _NT
    idx, loss_sum = pl.pallas_call(
        functools.partial(_argmin_body, nc_tiles),
        grid=(m_tiles, nc_tiles),
        in_specs=[
            pl.BlockSpec((_MT, k), lambda i, j: (i, 0)),
            pl.BlockSpec((_NT, k), lambda i, j: (j, 0)),
        ],
        out_specs=[
            pl.BlockSpec((_MT, 1), lambda i, j: (i, 0)),
            pl.BlockSpec(memory_space=pltpu.SMEM),
        ],
        out_shape=[
            jax.ShapeDtypeStruct((m, 1), jnp.int32),
            jax.ShapeDtypeStruct((1, 1), jnp.float32),
        ],
        scratch_shapes=[
            pltpu.VMEM((_MT, _FW), jnp.float32),
            pltpu.VMEM((_MT, _FW), jnp.float32),
            pltpu.VMEM((_MT, 1), jnp.float32),
            pltpu.VMEM((nc_tiles, _NT), jnp.float32),
            pltpu.VMEM((_MT, k), jnp.float32),
        ],
    )(flat, codebook)
    return idx.reshape(m), loss_sum[0, 0]


def _sc_gather(codebook, indices):
    t = indices.shape[0]
    k = codebook.shape[1]
    idx2 = indices.reshape(1, t)
    mesh = plsc.VectorSubcoreMesh(core_axis_name="core", subcore_axis_name="subcore")

    @functools.partial(
        pl.kernel,
        out_type=jax.ShapeDtypeStruct((t, k), codebook.dtype),
        mesh=mesh,
    )
    def gather_kernel(x_hbm, i_hbm, o_hbm):
        def body(i_vmem, o_vmem):
            pltpu.sync_copy(x_hbm.at[i_vmem.at[0]], o_vmem)

        pltpu.emit_pipeline(
            body,
            grid=(t // _GW,),
            in_specs=[pl.BlockSpec((1, _GW), index_map=lambda i: (0, i))],
            out_specs=[pl.BlockSpec((_GW, k), index_map=lambda i: (i, 0))],
            core_axis_name=("core", "subcore"),
            dimension_semantics=(pltpu.PARALLEL,),
        )(i_hbm, o_hbm)

    return gather_kernel(codebook, idx2)


def kernel(z_e, codebook):
    n, c, d, h, w = z_e.shape
    t = n * d * h * w
    flat = jnp.transpose(z_e, (0, 2, 3, 4, 1)).reshape(t, c)
    indices, loss_sum = _tc_argmin(flat, codebook)
    z_q_flat = _sc_gather(codebook, indices)
    vq_loss = 0.5 * loss_sum / (t * c)
    z_q = jnp.transpose(z_q_flat.reshape(n, d, h, w, c), (0, 4, 1, 2, 3))
    return (z_q, vq_loss, indices.reshape(n, d, h, w))


# speedup vs baseline: 1.2438x; 1.1480x over previous
import functools

import jax
import jax.numpy as jnp
from jax.experimental import pallas as pl
from jax.experimental.pallas import tpu as pltpu
from jax.experimental.pallas import tpu_sc as plsc

_MT = 4096
_NT = 1024
_GW = 128
_FW = 128


def _argmin_body(
    nc_tiles, x_ref, c_ref, idx_ref, loss_ref, minv_ref, mini_ref, xn_ref, cn_ref, x2_ref
):
    i = pl.program_id(0)
    j = pl.program_id(1)
    c = c_ref[...]

    @pl.when(j == 0)
    def _():
        x = x_ref[...]
        xn_ref[...] = jnp.sum(x * x, axis=1, keepdims=True)
        x2_ref[...] = x + x

    @pl.when(i == 0)
    def _():
        cn_ref[j, :] = jnp.sum(c * c, axis=1)

    xnorm = xn_ref[...]
    cnorm = cn_ref[j, :]
    prod2 = jax.lax.dot_general(
        x2_ref[...], c, (((1,), (1,)), ((), ())), preferred_element_type=jnp.float32
    )
    d = (xnorm + cnorm[None, :]) - prod2

    v = d
    off = None
    half = _NT // 2
    while half >= _FW:
        lo = v[:, :half]
        hi = v[:, half:]
        take = hi < lo
        if off is None:
            off = jnp.where(take, jnp.float32(half), jnp.float32(0.0))
        else:
            off = jnp.where(take, off[:, half:] + jnp.float32(half), off[:, :half])
        v = jnp.minimum(lo, hi)
        half //= 2
    base = off + jnp.float32(j * _NT)

    @pl.when(j == 0)
    def _():
        minv_ref[...] = v
        mini_ref[...] = base

    @pl.when(j != 0)
    def _():
        prev_v = minv_ref[...]
        take2 = v < prev_v
        minv_ref[...] = jnp.minimum(v, prev_v)
        mini_ref[...] = jnp.where(take2, base, mini_ref[...])

    @pl.when(j == nc_tiles - 1)
    def _():
        vv = minv_ref[...]
        best = jnp.min(vv, axis=1)
        full_idx = mini_ref[...] + jax.lax.broadcasted_iota(
            jnp.int32, (_MT, _FW), 1
        ).astype(jnp.float32)
        cand = jnp.where(vv == best[:, None], full_idx, jnp.float32(2.0**30))
        idx_ref[...] = jnp.min(cand, axis=1, keepdims=True).astype(jnp.int32)
        tile_loss = jnp.sum(best)
        prev = jnp.where(i == 0, 0.0, loss_ref[0, 0])
        loss_ref[0, 0] = prev + tile_loss


def _tc_argmin(flat, codebook):
    m, k = flat.shape
    nc = codebook.shape[0]
    m_tiles, nc_tiles = m // _MT, nc // _NT
    idx, loss_sum = pl.pallas_call(
        functools.partial(_argmin_body, nc_tiles),
        grid=(m_tiles, nc_tiles),
        in_specs=[
            pl.BlockSpec((_MT, k), lambda i, j: (i, 0)),
            pl.BlockSpec((_NT, k), lambda i, j: (j, 0)),
        ],
        out_specs=[
            pl.BlockSpec((_MT, 1), lambda i, j: (i, 0)),
            pl.BlockSpec(memory_space=pltpu.SMEM),
        ],
        out_shape=[
            jax.ShapeDtypeStruct((m, 1), jnp.int32),
            jax.ShapeDtypeStruct((1, 1), jnp.float32),
        ],
        scratch_shapes=[
            pltpu.VMEM((_MT, _FW), jnp.float32),
            pltpu.VMEM((_MT, _FW), jnp.float32),
            pltpu.VMEM((_MT, 1), jnp.float32),
            pltpu.VMEM((nc_tiles, _NT), jnp.float32),
            pltpu.VMEM((_MT, k), jnp.float32),
        ],
    )(flat, codebook)
    return idx.reshape(m), loss_sum[0, 0]


def _sc_gather(codebook, indices):
    t = indices.shape[0]
    k = codebook.shape[1]
    idx2 = indices.reshape(1, t)
    mesh = plsc.VectorSubcoreMesh(core_axis_name="core", subcore_axis_name="subcore")

    @functools.partial(
        pl.kernel,
        out_type=jax.ShapeDtypeStruct((t, k), codebook.dtype),
        mesh=mesh,
    )
    def gather_kernel(x_hbm, i_hbm, o_hbm):
        def body(i_vmem, o_vmem):
            pltpu.sync_copy(x_hbm.at[i_vmem.at[0]], o_vmem)

        pltpu.emit_pipeline(
            body,
            grid=(t // _GW,),
            in_specs=[pl.BlockSpec((1, _GW), index_map=lambda i: (0, i))],
            out_specs=[pl.BlockSpec((_GW, k), index_map=lambda i: (i, 0))],
            core_axis_name=("core", "subcore"),
            dimension_semantics=(pltpu.PARALLEL,),
        )(i_hbm, o_hbm)

    return gather_kernel(codebook, idx2)


def kernel(z_e, codebook):
    n, c, d, h, w = z_e.shape
    t = n * d * h * w
    flat = jnp.transpose(z_e, (0, 2, 3, 4, 1)).reshape(t, c)
    indices, loss_sum = _tc_argmin(flat, codebook)
    z_q_flat = _sc_gather(codebook, indices)
    vq_loss = 0.5 * loss_sum / (t * c)
    z_q = jnp.transpose(z_q_flat.reshape(n, d, h, w, c), (0, 4, 1, 2, 3))
    return (z_q, vq_loss, indices.reshape(n, d, h, w))
